# baseline (speedup 1.0000x reference)
import functools

import jax
import jax.numpy as jnp
from jax import lax
from jax.experimental import pallas as pl
from jax.experimental.pallas import tpu as pltpu
from jax.experimental.pallas import tpu_sc as plsc

NC = 2
NS = 16
NW = NC * NS
CHUNK = 128
NBUF = 2
IB = 8


@functools.lru_cache(maxsize=None)
def _sc_partial_sums(n_nodes: int, d: int, n_chunks: int):
    acc_rows = -(-(n_nodes + 1) // (NS * CHUNK)) * (NS * CHUNK)
    zero_chunks_per_tile = acc_rows // NS // CHUNK
    out_rows_per_tile = acc_rows // NS
    assert d % 16 == 0

    mesh = plsc.VectorSubcoreMesh(core_axis_name="c", subcore_axis_name="s")

    @functools.partial(
        pl.kernel,
        out_type=jax.ShapeDtypeStruct((NC, acc_rows, d), jnp.float32),
        mesh=mesh,
        scratch_types=[
            pltpu.VMEM((2, IB, CHUNK), jnp.int32),
            pltpu.VMEM((n_chunks, CHUNK), jnp.int32),
            pltpu.VMEM((NBUF, CHUNK, d), jnp.float32),
            pltpu.VMEM_SHARED((acc_rows, d), jnp.float32),
            [pltpu.SemaphoreType.DMA] * NBUF,
            [pltpu.SemaphoreType.DMA] * 2,
        ],
    )
    def sc_kernel(x_hbm, col_hbm, row_hbm, out_hbm, colb, row_v, gbufs, acc,
                  gsems, isems):
        cid = lax.axis_index("c")
        sid = lax.axis_index("s")
        wid = cid * NS + sid

        pltpu.sync_copy(row_hbm.at[wid], row_v)

        zbuf = gbufs.at[0]
        def zero_body(i, carry):
            for j in range(d // 16):
                zbuf[i, pl.ds(j * 16, 16)] = jnp.zeros((16,), jnp.float32)
            return carry
        lax.fori_loop(0, CHUNK, zero_body, 0)
        for k in range(zero_chunks_per_tile):
            pltpu.sync_copy(
                zbuf, acc.at[pl.ds((sid * zero_chunks_per_tile + k) * CHUNK, CHUNK)]
            )
        plsc.subcore_barrier()

        nblk = n_chunks // IB
        assert n_chunks % (2 * IB) == 0 and IB % NBUF == 0 and nblk >= 2

        def col_copy(bi, s):
            return pltpu.make_async_copy(
                col_hbm.at[wid, pl.ds(bi * IB, IB)], colb.at[s], isems[s]
            )

        def gather(s, k, b):
            return pltpu.make_async_copy(
                x_hbm.at[colb.at[s, k]], gbufs.at[b], gsems[b]
            )

        col_copy(0, 0).start()
        col_copy(0, 0).wait()
        col_copy(1, 1).start()
        gather(0, 0, 0).start()
        gather(0, 1, 1).start()

        def pair_body(p, carry):
            for s in range(2):
                bi = p * 2 + s
                for k in range(IB):
                    j = bi * IB + k
                    b = k % NBUF
                    gather(s, k, b).wait()
                    pltpu.sync_copy(gbufs.at[b], acc.at[row_v.at[j]], add=True)
                    if k + NBUF < IB:
                        @pl.when(j + NBUF < n_chunks)
                        def _():
                            gather(s, k + NBUF, b).start()
                    else:
                        if k + NBUF == IB:
                            @pl.when(j + NBUF < n_chunks)
                            def _():
                                col_copy(bi + 1, 1 - s).wait()

                        @pl.when(j + NBUF < n_chunks)
                        def _():
                            gather(1 - s, k + NBUF - IB, b).start()

                @pl.when(bi + 2 < nblk)
                def _():
                    col_copy(bi + 2, s).start()
            return carry
        lax.fori_loop(0, nblk // 2, pair_body, 0)
        plsc.subcore_barrier()

        pltpu.sync_copy(
            acc.at[pl.ds(sid * out_rows_per_tile, out_rows_per_tile)],
            out_hbm.at[cid, pl.ds(sid * out_rows_per_tile, out_rows_per_tile)],
        )

    return sc_kernel


@functools.lru_cache(maxsize=None)
def _combine(n_nodes: int, d: int):
    blk = 1000
    assert n_nodes % blk == 0

    def body(x_ref, a_ref, b_ref, o_ref):
        o_ref[:, :d] = x_ref[...]
        o_ref[:, d:] = a_ref[...] + b_ref[...]

    return pl.pallas_call(
        body,
        grid=(n_nodes // blk,),
        in_specs=[pl.BlockSpec((blk, d), lambda i: (i, 0))] * 3,
        out_specs=pl.BlockSpec((blk, 2 * d), lambda i: (i, 0)),
        out_shape=jax.ShapeDtypeStruct((n_nodes, 2 * d), jnp.float32),
    )


def kernel(x, edge_index):
    n_nodes, d = x.shape
    n_edges = edge_index.shape[1]
    ei = edge_index.astype(jnp.int32)
    row, col = ei[0], ei[1]

    per_round = NW * CHUNK
    n_chunks = -(-(-(-n_edges // per_round)) // (2 * IB)) * (2 * IB)
    e_pad = n_chunks * per_round
    if e_pad != n_edges:
        pad = e_pad - n_edges
        row = jnp.concatenate([row, jnp.full((pad,), n_nodes, jnp.int32)])
        col = jnp.concatenate([col, jnp.zeros((pad,), jnp.int32)])
    row3 = row.reshape(NW, n_chunks, CHUNK)
    col3 = col.reshape(NW, n_chunks, CHUNK)

    partial = _sc_partial_sums(n_nodes, d, n_chunks)(x, col3, row3)
    return _combine(n_nodes, d)(x, partial[0, :n_nodes], partial[1, :n_nodes])
